# baseline (speedup 1.0000x reference)
import jax
import jax.numpy as jnp
from jax import lax
from jax.experimental import pallas as pl

_N = 10000
_H = 128
_M = 64

_DN_T = (((1,), (1,)), ((), ()))


def _dot_t(a, b):
    return lax.dot_general(a, b, _DN_T, preferred_element_type=jnp.float32)


def _fused_body(x_ref, win_ref, bin_ref, mem_ref,
                l0_wih, l0_whh, l0_bih, l0_bhh, l0_wm1, l0_bm1, l0_wm2,
                l0_bm2, l0_wa, l0_ba,
                l1_wih, l1_whh, l1_bih, l1_bhh, l1_wm1, l1_bm1, l1_wm2,
                l1_bm2, l1_wa, l1_ba,
                wc1_ref, bc1_ref, wc2_ref, bc2_ref, out_ref):
    h = _dot_t(x_ref[...], win_ref[...]) + bin_ref[...]
    mem = mem_ref[...]
    for (wih, whh, bih, bhh, wm1, bm1, wm2, bm2, wa, ba) in (
            (l0_wih, l0_whh, l0_bih, l0_bhh, l0_wm1, l0_bm1, l0_wm2,
             l0_bm2, l0_wa, l0_ba),
            (l1_wih, l1_whh, l1_bih, l1_bhh, l1_wm1, l1_bm1, l1_wm2,
             l1_bm2, l1_wa, l1_ba)):
        xm = jnp.sum(h, axis=0, keepdims=True) * (1.0 / _N)
        gi = _dot_t(xm, wih[...]) + bih[...]
        gh = _dot_t(mem, whh[...]) + bhh[...]
        r = jax.nn.sigmoid(gi[:, 0:_M] + gh[:, 0:_M])
        z = jax.nn.sigmoid(gi[:, _M:2 * _M] + gh[:, _M:2 * _M])
        n = jnp.tanh(gi[:, 2 * _M:] + r * gh[:, 2 * _M:])
        mem = (1.0 - z) * n + z * mem
        c = _dot_t(mem, wm1[:, _H:]) + bm1[...]
        u = jnp.maximum(_dot_t(h, wm1[:, 0:_H]) + c, 0.0)
        msg = _dot_t(u, wm2[...]) + bm2[...]
        h = _dot_t(msg, wa[...]) + ba[...]
    v = jnp.maximum(_dot_t(h, wc1_ref[...]) + bc1_ref[...], 0.0)
    out_ref[...] = _dot_t(v, wc2_ref[...]) + bc2_ref[...]


def kernel(x, edge_index, edge_attr, t, W_in, b_in, memory,
           l0_wih, l0_whh, l0_bih, l0_bhh, l0_Wm1, l0_bm1, l0_Wm2, l0_bm2,
           l0_Wa, l0_ba,
           l1_wih, l1_whh, l1_bih, l1_bhh, l1_Wm1, l1_bm1, l1_Wm2, l1_bm2,
           l1_Wa, l1_ba,
           Wc1, bc1, Wc2, bc2):
    del edge_index, edge_attr, t
    return pl.pallas_call(
        _fused_body,
        out_shape=jax.ShapeDtypeStruct((_N, 2), jnp.float32),
    )(x, W_in, b_in, memory,
      l0_wih, l0_whh, l0_bih, l0_bhh, l0_Wm1, l0_bm1, l0_Wm2, l0_bm2,
      l0_Wa, l0_ba,
      l1_wih, l1_whh, l1_bih, l1_bhh, l1_Wm1, l1_bm1, l1_Wm2, l1_bm2,
      l1_Wa, l1_ba,
      Wc1, bc1, Wc2, bc2)


# speedup vs baseline: 2.0537x; 2.0537x over previous
import jax
import jax.numpy as jnp
from jax import lax
from jax.experimental import pallas as pl

_N = 10000
_H = 128
_M = 64

_DN_T = (((1,), (1,)), ((), ()))


def _dot_t(a, b):
    return lax.dot_general(a, b, _DN_T, preferred_element_type=jnp.float32)


def _dot(a, b):
    return jnp.dot(a, b, preferred_element_type=jnp.float32)


def _fused_body(x_ref, win_ref, bin_ref, mem_ref,
                l0_wih, l0_whh_t, l0_bih, l0_bhh, l0_wm1_t, l0_bm1, l0_wm2,
                l0_bm2, l0_wa, l0_ba,
                l1_wih, l1_whh_t, l1_bih, l1_bhh, l1_wm1_t, l1_bm1, l1_wm2,
                l1_bm2, l1_wa, l1_ba,
                wc1_ref, bc1_ref, wc2_ref, bc2_ref, out_ref):
    h = _dot_t(x_ref[...], win_ref[...]) + bin_ref[...]
    mem = mem_ref[...]
    for (wih, whh_t, bih, bhh, wm1_t, bm1, wm2, bm2, wa, ba) in (
            (l0_wih, l0_whh_t, l0_bih, l0_bhh, l0_wm1_t, l0_bm1, l0_wm2,
             l0_bm2, l0_wa, l0_ba),
            (l1_wih, l1_whh_t, l1_bih, l1_bhh, l1_wm1_t, l1_bm1, l1_wm2,
             l1_bm2, l1_wa, l1_ba)):
        xm = jnp.sum(h, axis=0, keepdims=True) * (1.0 / _N)
        gi_r = _dot_t(xm, wih[0:_M, :]) + bih[0:_M]
        gi_z = _dot_t(xm, wih[_M:2 * _M, :]) + bih[_M:2 * _M]
        gi_n = _dot_t(xm, wih[2 * _M:, :]) + bih[2 * _M:]
        gh = _dot(mem, whh_t[...]) + bhh[...]
        r = jax.nn.sigmoid(gi_r + gh[:, 0:_M])
        z = jax.nn.sigmoid(gi_z + gh[:, _M:2 * _M])
        n = jnp.tanh(gi_n + r * gh[:, 2 * _M:])
        mem = (1.0 - z) * n + z * mem
        c = _dot(mem, wm1_t[_H:, :]) + bm1[...]
        u = jnp.maximum(_dot(h, wm1_t[0:_H, :]) + c, 0.0)
        msg = _dot_t(u, wm2[...]) + bm2[...]
        h = _dot_t(msg, wa[...]) + ba[...]
    v = jnp.maximum(_dot_t(h, wc1_ref[...]) + bc1_ref[...], 0.0)
    out_ref[...] = (_dot_t(wc2_ref[...], v)
                    + jnp.expand_dims(bc2_ref[...], 1))


def kernel(x, edge_index, edge_attr, t, W_in, b_in, memory,
           l0_wih, l0_whh, l0_bih, l0_bhh, l0_Wm1, l0_bm1, l0_Wm2, l0_bm2,
           l0_Wa, l0_ba,
           l1_wih, l1_whh, l1_bih, l1_bhh, l1_Wm1, l1_bm1, l1_Wm2, l1_bm2,
           l1_Wa, l1_ba,
           Wc1, bc1, Wc2, bc2):
    del edge_index, edge_attr, t
    out_t = pl.pallas_call(
        _fused_body,
        out_shape=jax.ShapeDtypeStruct((2, _N), jnp.float32),
    )(x, W_in, b_in, memory,
      l0_wih, l0_whh.T, l0_bih, l0_bhh, l0_Wm1.T, l0_bm1, l0_Wm2, l0_bm2,
      l0_Wa, l0_ba,
      l1_wih, l1_whh.T, l1_bih, l1_bhh, l1_Wm1.T, l1_bm1, l1_Wm2, l1_bm2,
      l1_Wa, l1_ba,
      Wc1, bc1, Wc2, bc2)
    return out_t.T
